# baseline (speedup 1.0000x reference)
import functools

import jax
import jax.numpy as jnp
from jax import lax
from jax.experimental import pallas as pl
from jax.experimental.pallas import tpu as pltpu
from jax.experimental.pallas import tpu_sc as plsc

_LANES = 16
_GATHER_CHUNK = 128


@functools.cache
def _make_sc_kernel(batch: int, n_factors: int):
    info = plsc.get_sparse_core_info()
    num_workers = info.num_cores * info.num_subcores
    b_per_w = batch // num_workers
    assert b_per_w * num_workers == batch
    n_chunks = b_per_w // _GATHER_CHUNK
    n_blocks = b_per_w // _LANES

    mesh = plsc.VectorSubcoreMesh(core_axis_name="c", subcore_axis_name="s")

    @functools.partial(
        pl.kernel,
        mesh=mesh,
        out_type=jax.ShapeDtypeStruct((batch,), jnp.float32),
        scratch_types=[
            pltpu.VMEM((b_per_w,), jnp.int32),
            pltpu.VMEM((b_per_w,), jnp.int32),
            pltpu.VMEM((b_per_w, n_factors), jnp.float32),
            pltpu.VMEM((b_per_w, n_factors), jnp.float32),
            pltpu.VMEM((b_per_w,), jnp.float32),
            pltpu.VMEM((_LANES * _LANES,), jnp.float32),
            pltpu.SemaphoreType.DMA,
        ],
        compiler_params=pltpu.CompilerParams(
            needs_layout_passes=False, use_tc_tiling_on_sc=False),
    )
    def sc_kernel(user_hbm, item_hbm, uemb_hbm, iemb_hbm, out_hbm,
                  idx_u, idx_i, rows_u, rows_i, out_v, tbuf, sem):
        wid = lax.axis_index("s") * info.num_cores + lax.axis_index("c")
        base = wid * b_per_w

        pltpu.sync_copy(user_hbm.at[pl.ds(base, b_per_w)], idx_u)
        pltpu.sync_copy(item_hbm.at[pl.ds(base, b_per_w)], idx_i)

        copies = []
        for j in range(n_chunks):
            sl = pl.ds(j * _GATHER_CHUNK, _GATHER_CHUNK)
            copies.append(
                pltpu.async_copy(uemb_hbm.at[idx_u.at[sl]], rows_u.at[sl], sem))
            copies.append(
                pltpu.async_copy(iemb_hbm.at[idx_i.at[sl]], rows_i.at[sl], sem))
        for c in copies:
            c.wait()

        iota = lax.iota(jnp.int32, _LANES)

        def blk_body(blk, carry):
            for r in range(_LANES):
                row = blk * _LANES + r
                u0 = rows_u[row, pl.ds(0, _LANES)]
                u1 = rows_u[row, pl.ds(_LANES, _LANES)]
                v0 = rows_i[row, pl.ds(0, _LANES)]
                v1 = rows_i[row, pl.ds(_LANES, _LANES)]
                p = u0 * v0 + u1 * v1
                plsc.store_scatter(tbuf, [iota * _LANES + r], p)
            acc = tbuf[pl.ds(0, _LANES)]
            for l in range(1, _LANES):
                acc = acc + tbuf[pl.ds(l * _LANES, _LANES)]
            out_v[pl.ds(blk * _LANES, _LANES)] = acc
            return carry

        lax.fori_loop(0, n_blocks, blk_body, 0)
        pltpu.sync_copy(out_v, out_hbm.at[pl.ds(base, b_per_w)])

    return sc_kernel


@jax.jit
def kernel(user, item, user_emb, item_emb):
    sc = _make_sc_kernel(user.shape[0], user_emb.shape[1])
    return sc(user.astype(jnp.int32), item.astype(jnp.int32),
              user_emb, item_emb)
